# Initial kernel scaffold; baseline (speedup 1.0000x reference)
import jax
import jax.numpy as jnp
from jax.experimental import pallas as pl


def kernel(z, Wd, Wu, E):
    raise NotImplementedError("write your pallas kernel here")


# speedup vs baseline: 1.7191x; 1.7191x over previous
import jax
import jax.numpy as jnp
from jax.experimental import pallas as pl

_B, _SEQ, _C, _H = 32, 4096, 192, 4
_W = _SEQ // _H
_FIX = 384
_OVL = 4
_NVQ = 6
_K = 1024
_VD = 256
_COMMIT = 0.25
_NROW = _B * _W // _OVL
_RB = 512


def _down_kernel(z_ref, wd_ref, out_ref):
    acc = jnp.zeros((_W, _FIX), jnp.float32)
    for h in range(_H):
        acc = acc + jnp.dot(z_ref[0, h], wd_ref[h],
                            preferred_element_type=jnp.float32)
    out_ref[0] = acc


def _vq_kernel(zo_ref, e_ref, et_ref, zq_ref, loss_ref):
    total = jnp.float32(0.0)
    for i in range(_NVQ):
        zf = zo_ref[:, i * _VD:(i + 1) * _VD]
        ei = e_ref[i]
        z2 = jnp.sum(zf * zf, axis=1, keepdims=True)
        e2 = jnp.sum(ei * ei, axis=0, keepdims=True)
        dist = z2 - 2.0 * jnp.dot(zf, ei, preferred_element_type=jnp.float32) + e2
        idx = jnp.argmin(dist, axis=1)
        onehot = (jax.lax.broadcasted_iota(jnp.int32, (_RB, _K), 1)
                  == idx[:, None]).astype(jnp.float32)
        zq = jnp.dot(onehot, et_ref[i], preferred_element_type=jnp.float32)
        d = zq - zf
        total = total + jnp.sum(d * d)
        zq_ref[:, i * _VD:(i + 1) * _VD] = zq

    @pl.when(pl.program_id(0) == 0)
    def _init():
        loss_ref[...] = jnp.full((8, 128), total, jnp.float32)

    @pl.when(pl.program_id(0) != 0)
    def _acc():
        loss_ref[...] = loss_ref[...] + jnp.full((8, 128), total, jnp.float32)


def _up_kernel(zq_ref, wu_ref, out_ref):
    x = zq_ref[0]
    for h in range(_H):
        out_ref[0, h] = jnp.dot(x, wu_ref[h], preferred_element_type=jnp.float32)


def kernel(z, Wd, Wu, E):
    z4 = z.reshape(_B, _H, _W, _C)
    wd = Wd.reshape(_C, _H, _FIX).transpose(1, 0, 2)
    zp = pl.pallas_call(
        _down_kernel,
        grid=(_B,),
        in_specs=[pl.BlockSpec((1, _H, _W, _C), lambda b: (b, 0, 0, 0)),
                  pl.BlockSpec((_H, _C, _FIX), lambda b: (0, 0, 0))],
        out_specs=pl.BlockSpec((1, _W, _FIX), lambda b: (b, 0, 0)),
        out_shape=jax.ShapeDtypeStruct((_B, _W, _FIX), jnp.float32),
    )(z4, wd)

    zo = zp.reshape(_NROW, _OVL * _FIX)
    et = E.transpose(0, 2, 1)
    nblk = _NROW // _RB
    zq, lossb = pl.pallas_call(
        _vq_kernel,
        grid=(nblk,),
        in_specs=[pl.BlockSpec((_RB, _OVL * _FIX), lambda r: (r, 0)),
                  pl.BlockSpec((_NVQ, _VD, _K), lambda r: (0, 0, 0)),
                  pl.BlockSpec((_NVQ, _K, _VD), lambda r: (0, 0, 0))],
        out_specs=[pl.BlockSpec((_RB, _OVL * _FIX), lambda r: (r, 0)),
                   pl.BlockSpec((8, 128), lambda r: (0, 0))],
        out_shape=[jax.ShapeDtypeStruct((_NROW, _OVL * _FIX), jnp.float32),
                   jax.ShapeDtypeStruct((8, 128), jnp.float32)],
    )(zo, E, et)

    zqp = zq.reshape(_B, _W, _FIX)
    wu = Wu.reshape(_FIX, _C, _H).transpose(2, 0, 1)
    out = pl.pallas_call(
        _up_kernel,
        grid=(_B,),
        in_specs=[pl.BlockSpec((1, _W, _FIX), lambda b: (b, 0, 0)),
                  pl.BlockSpec((_H, _FIX, _C), lambda b: (0, 0, 0))],
        out_specs=pl.BlockSpec((1, _H, _W, _C), lambda b: (b, 0, 0, 0)),
        out_shape=jax.ShapeDtypeStruct((_B, _H, _W, _C), jnp.float32),
    )(zqp, wu)

    zq_out = out.reshape(_B, _SEQ, _C)
    loss = lossb[0, 0] * (_COMMIT / (_NROW * _VD * _NVQ))
    return zq_out, loss


# speedup vs baseline: 1.7688x; 1.0289x over previous
import jax
import jax.numpy as jnp
from jax.experimental import pallas as pl

_B, _SEQ, _C, _H = 32, 4096, 192, 4
_W = _SEQ // _H
_FIX = 384
_OVL = 4
_NVQ = 6
_K = 1024
_VD = 256
_COMMIT = 0.25
_NROW = _B * _W // _OVL
_RB = 512


def _down_kernel(z_ref, wd_ref, out_ref):
    acc = jnp.zeros((_W, _FIX), jnp.float32)
    for h in range(_H):
        acc = acc + jnp.dot(z_ref[0, h], wd_ref[h],
                            preferred_element_type=jnp.float32)
    out_ref[0] = acc


def _vq_kernel(zo_ref, e_ref, et_ref, zq_ref, loss_ref):
    total = jnp.float32(0.0)
    for i in range(_NVQ):
        zf = zo_ref[:, i * _VD:(i + 1) * _VD]
        ei = e_ref[i]
        z2 = jnp.sum(zf * zf, axis=1, keepdims=True)
        e2 = jnp.sum(ei * ei, axis=0, keepdims=True)
        dist = z2 - 2.0 * jnp.dot(zf, ei, preferred_element_type=jnp.float32) + e2
        idx = jnp.argmin(dist, axis=1)
        total = total + jnp.sum(jnp.min(dist, axis=1))
        onehot = (jax.lax.broadcasted_iota(jnp.int32, (_RB, _K), 1)
                  == idx[:, None]).astype(jnp.bfloat16)
        zq_ref[:, i * _VD:(i + 1) * _VD] = jnp.dot(
            onehot, et_ref[i],
            preferred_element_type=jnp.float32).astype(jnp.bfloat16)

    @pl.when(pl.program_id(0) == 0)
    def _init():
        loss_ref[...] = jnp.full((8, 128), total, jnp.float32)

    @pl.when(pl.program_id(0) != 0)
    def _acc():
        loss_ref[...] = loss_ref[...] + jnp.full((8, 128), total, jnp.float32)


def _up_kernel(zq_ref, wu_ref, out_ref):
    x = zq_ref[0]
    for h in range(_H):
        out_ref[0, h] = jnp.dot(x, wu_ref[h], preferred_element_type=jnp.float32)


def kernel(z, Wd, Wu, E):
    z4 = z.reshape(_B, _H, _W, _C)
    wd = Wd.reshape(_C, _H, _FIX).transpose(1, 0, 2)
    zp = pl.pallas_call(
        _down_kernel,
        grid=(_B,),
        in_specs=[pl.BlockSpec((1, _H, _W, _C), lambda b: (b, 0, 0, 0)),
                  pl.BlockSpec((_H, _C, _FIX), lambda b: (0, 0, 0))],
        out_specs=pl.BlockSpec((1, _W, _FIX), lambda b: (b, 0, 0)),
        out_shape=jax.ShapeDtypeStruct((_B, _W, _FIX), jnp.float32),
    )(z4, wd)

    zo = zp.reshape(_NROW, _OVL * _FIX)
    et = E.transpose(0, 2, 1).astype(jnp.bfloat16)
    nblk = _NROW // _RB
    zq, lossb = pl.pallas_call(
        _vq_kernel,
        grid=(nblk,),
        in_specs=[pl.BlockSpec((_RB, _OVL * _FIX), lambda r: (r, 0)),
                  pl.BlockSpec((_NVQ, _VD, _K), lambda r: (0, 0, 0)),
                  pl.BlockSpec((_NVQ, _K, _VD), lambda r: (0, 0, 0))],
        out_specs=[pl.BlockSpec((_RB, _OVL * _FIX), lambda r: (r, 0)),
                   pl.BlockSpec((8, 128), lambda r: (0, 0))],
        out_shape=[jax.ShapeDtypeStruct((_NROW, _OVL * _FIX), jnp.bfloat16),
                   jax.ShapeDtypeStruct((8, 128), jnp.float32)],
    )(zo, E, et)

    zqp = zq.reshape(_B, _W, _FIX)
    wu = Wu.reshape(_FIX, _C, _H).transpose(2, 0, 1).astype(jnp.bfloat16)
    out = pl.pallas_call(
        _up_kernel,
        grid=(_B,),
        in_specs=[pl.BlockSpec((1, _W, _FIX), lambda b: (b, 0, 0)),
                  pl.BlockSpec((_H, _FIX, _C), lambda b: (0, 0, 0))],
        out_specs=pl.BlockSpec((1, _H, _W, _C), lambda b: (b, 0, 0, 0)),
        out_shape=jax.ShapeDtypeStruct((_B, _H, _W, _C), jnp.float32),
    )(zqp, wu)

    zq_out = out.reshape(_B, _SEQ, _C)
    loss = lossb[0, 0] * (_COMMIT / (_NROW * _VD * _NVQ))
    return zq_out, loss
